# Initial kernel scaffold; baseline (speedup 1.0000x reference)
import jax
import jax.numpy as jnp
from jax.experimental import pallas as pl


def kernel(x, token_table, pos_table):
    raise NotImplementedError("write your pallas kernel here")


# speedup vs baseline: 3.2476x; 3.2476x over previous
import functools

import jax
import jax.numpy as jnp
from jax import lax
from jax.experimental import pallas as pl
from jax.experimental.pallas import tpu as pltpu
from jax.experimental.pallas import tpu_sc as plsc

_INFO = plsc.get_sparse_core_info()
_NC = _INFO.num_cores
_NS = _INFO.num_subcores
_NW = _NC * _NS
_LANES = _INFO.num_lanes

_NBUF = 4
_PF = _NBUF - 1


@functools.lru_cache(maxsize=None)
def _build(B, L, D, V):
    assert B % _NW == 0 and D % _LANES == 0
    seq_per_w = B // _NW
    assert seq_per_w % _NBUF == 0
    c0 = min(128, L)
    c1 = L - c0
    nvec = D // _LANES

    mesh = plsc.VectorSubcoreMesh(core_axis_name="c", subcore_axis_name="s")

    @functools.partial(
        pl.kernel,
        out_type=jax.ShapeDtypeStruct((B, L, D), jnp.float32),
        mesh=mesh,
        compiler_params=pltpu.CompilerParams(use_tc_tiling_on_sc=False),
        scratch_types=[
            pltpu.VMEM((seq_per_w, L), jnp.int32),
            pltpu.VMEM((L, D), jnp.float32),
            [pltpu.VMEM((L, D), jnp.float32)] * _NBUF,
            [pltpu.SemaphoreType.DMA] * _NBUF,
            [pltpu.SemaphoreType.DMA] * _NBUF,
        ],
    )
    def emb_kernel(x_hbm, tok_hbm, pos_hbm, out_hbm, idx_v, pos_v,
                   tok_bufs, gsems, osems):
        cid = lax.axis_index("c")
        sid = lax.axis_index("s")
        wid = sid * _NC + cid
        seq0 = wid * seq_per_w

        pltpu.sync_copy(x_hbm.at[pl.ds(seq0, seq_per_w)], idx_v)
        pltpu.sync_copy(pos_hbm, pos_v)

        def start_gather(s, b):
            tb = tok_bufs[b]
            pltpu.async_copy(tok_hbm.at[idx_v.at[s, pl.ds(0, c0)]],
                             tb.at[pl.ds(0, c0)], gsems[b])
            if c1:
                pltpu.async_copy(tok_hbm.at[idx_v.at[s, pl.ds(c0, c1)]],
                                 tb.at[pl.ds(c0, c1)], gsems[b])

        def wait_gather(b):
            tb = tok_bufs[b]
            pltpu.make_async_copy(tok_hbm.at[idx_v.at[0, pl.ds(0, c0)]],
                                  tb.at[pl.ds(0, c0)], gsems[b]).wait()
            if c1:
                pltpu.make_async_copy(tok_hbm.at[idx_v.at[0, pl.ds(c0, c1)]],
                                      tb.at[pl.ds(c0, c1)], gsems[b]).wait()

        def start_out(s, b):
            pltpu.async_copy(tok_bufs[b], out_hbm.at[seq0 + s], osems[b])

        def wait_out(b):
            pltpu.make_async_copy(tok_bufs[b], out_hbm.at[seq0], osems[b]).wait()

        def add_pos(b):
            tb = tok_bufs[b]

            @pl.loop(0, L, unroll=2)
            def _(r):
                for j in range(nvec):
                    sl = pl.ds(j * _LANES, _LANES)
                    plsc.addupdate(tb.at[r, sl], pos_v[r, sl])

        for s in range(_PF):
            start_gather(s, s % _NBUF)

        @pl.loop(0, seq_per_w, step=_NBUF)
        def _(g):
            for b in range(_NBUF):
                s = g + b
                wait_gather(b)
                add_pos(b)
                start_out(s, b)
                sp = s + _PF
                bp = (b + _PF) % _NBUF

                @pl.when(sp < seq_per_w)
                def _():
                    @pl.when(sp >= _NBUF)
                    def _():
                        wait_out(bp)

                    start_gather(sp, bp)

        for b in range(_NBUF):
            wait_out(b)

    return emb_kernel


def kernel(x, token_table, pos_table):
    B, L = x.shape
    V, D = token_table.shape
    fn = _build(B, L, D, V)
    return fn(x.astype(jnp.int32), token_table, pos_table)
